# Initial kernel scaffold; baseline (speedup 1.0000x reference)
import jax
import jax.numpy as jnp
from jax.experimental import pallas as pl


def kernel(X):
    raise NotImplementedError("write your pallas kernel here")


# speedup vs baseline: 12.9721x; 12.9721x over previous
import functools

import jax
import jax.numpy as jnp
from jax import lax
from jax.experimental import pallas as pl
from jax.experimental.pallas import tpu as pltpu
from jax.experimental.pallas import tpu_sc as plsc

TOPK = 256
ROWS = 128
COLS = 32768
CHUNK = 128
NCH = COLS // CHUNK
RBLK = 16

NWORKERS = 32
RPW = ROWS // NWORKERS

def _tc_body(x_ref, p_ref):
    x = x_ref[...]
    b = lax.bitcast_convert_type(x, jnp.int32)
    k = jnp.where(b < 0, b ^ jnp.int32(0x7FFFFFFF), b)

    kk = jnp.int32(TOPK)
    cnt0 = jnp.sum((k >= 0).astype(jnp.int32), axis=(1, 2), keepdims=True)
    base = jnp.where(cnt0 >= kk, jnp.int32(0), jnp.int32(-2147483648))

    def step(t, P):
        bit = jnp.int32(1) << (jnp.int32(30) - t)
        cand = base + (P | bit)
        cnt = jnp.sum((k >= cand).astype(jnp.int32), axis=(1, 2),
                      keepdims=True)
        return jnp.where(cnt >= kk, P | bit, P)

    P = lax.fori_loop(0, 31, step, jnp.zeros_like(base))
    T = base + P

    m_gt = (k > T).astype(jnp.float32)
    m_eq = (k == T).astype(jnp.float32)
    c_gt = jnp.sum(m_gt, axis=(1, 2), keepdims=True)

    i128 = lax.broadcasted_iota(jnp.int32, (CHUNK, CHUNK), 0)
    j128 = lax.broadcasted_iota(jnp.int32, (CHUNK, CHUNK), 1)
    tri_incl = (i128 <= j128).astype(jnp.float32)
    inch = lax.broadcasted_iota(jnp.int32, (NCH, NCH), 0)
    jnch = lax.broadcasted_iota(jnp.int32, (NCH, NCH), 1)
    tri_excl = (inch < jnch).astype(jnp.float32)

    def row_cumsum(m):
        m2 = m.reshape(RBLK * NCH, CHUNK)
        cc = lax.dot_general(m2, tri_incl, (((1,), (0,)), ((), ())),
                             preferred_element_type=jnp.float32)
        cc = cc.reshape(RBLK, NCH, CHUNK)
        tot = cc[:, :, CHUNK - 1]
        pre = lax.dot_general(tot, tri_excl, (((1,), (0,)), ((), ())),
                              preferred_element_type=jnp.float32)
        return cc + pre[:, :, None]

    p_gt = row_cumsum(m_gt)
    p_eq = row_cumsum(m_eq)
    quota = jnp.float32(TOPK) - c_gt
    p_sel = p_gt + jnp.minimum(p_eq, quota)
    p_ref[...] = p_sel.astype(jnp.int32)


def _tc_stage(x3):
    return pl.pallas_call(
        _tc_body,
        grid=(ROWS // RBLK,),
        in_specs=[pl.BlockSpec((RBLK, NCH, CHUNK), lambda i: (i, 0, 0))],
        out_specs=pl.BlockSpec((RBLK, NCH, CHUNK), lambda i: (i, 0, 0)),
        out_shape=jax.ShapeDtypeStruct((ROWS, NCH, CHUNK), jnp.int32),
    )(x3)


def _sc_stage(p_sel, x_flat):
    mesh = plsc.VectorSubcoreMesh(core_axis_name="c", subcore_axis_name="s")

    @functools.partial(
        pl.kernel,
        mesh=mesh,
        compiler_params=pltpu.CompilerParams(needs_layout_passes=False),
        out_type=jax.ShapeDtypeStruct((ROWS, 2, 128), jnp.float32),
        scratch_types=[
            pltpu.VMEM((COLS,), jnp.int32),
            pltpu.VMEM((2, 128), jnp.int32),
            pltpu.VMEM((2, 128), jnp.float32),
            pltpu.SemaphoreType.DMA,
        ],
    )
    def sc_kernel(p_hbm, x_hbm, out_hbm, p_v, idx_v, val_v, sem):
        wid = lax.axis_index("s") * 2 + lax.axis_index("c")
        for rr in range(RPW):
            r = wid * RPW + rr
            pltpu.sync_copy(p_hbm.at[r], p_v)
            rbase = r * COLS

            for h in range(2):
                def jbody(j, _, h=h):
                    t = (h * 128 + j * 16 + 1) + lax.iota(jnp.int32, 16)
                    pos = jnp.zeros(16, jnp.int32)
                    for sbit in range(14, -1, -1):
                        s = jnp.int32(1 << sbit)
                        probe = pos + (s - 1)
                        v = plsc.load_gather(p_v, [probe])
                        pos = pos + jnp.where(v < t, s, jnp.int32(0))
                    idx_v[h, pl.ds(j * 16, 16)] = rbase + pos
                    return 0

                lax.fori_loop(0, 8, jbody, 0)

            for h in range(2):
                pltpu.async_copy(x_hbm.at[idx_v.at[h]], val_v.at[h],
                                 sem).wait()
            pltpu.sync_copy(val_v, out_hbm.at[r])

    return sc_kernel(p_sel, x_flat)


@jax.jit
def kernel(X):
    x3 = X.reshape(ROWS, NCH, CHUNK)
    p_sel = _tc_stage(x3)
    out = _sc_stage(p_sel.reshape(ROWS, COLS), X.reshape(-1))
    return out.reshape(ROWS, TOPK)


# speedup vs baseline: 14.1554x; 1.0912x over previous
import functools

import jax
import jax.numpy as jnp
from jax import lax
from jax.experimental import pallas as pl
from jax.experimental.pallas import tpu as pltpu
from jax.experimental.pallas import tpu_sc as plsc

TOPK = 256
ROWS = 128
COLS = 32768
CHUNK = 128
NCH = COLS // CHUNK
RBLK = 16

NWORKERS = 32
RPW = ROWS // NWORKERS

def _tc_body(x_ref, p_ref, xo_ref):
    x = x_ref[...]
    b = lax.bitcast_convert_type(x, jnp.int32)
    k = jnp.where(b < 0, b ^ jnp.int32(0x7FFFFFFF), b)

    kk = jnp.int32(TOPK)
    cnt0 = jnp.sum((k >= 0).astype(jnp.int32), axis=(1, 2), keepdims=True)
    base = jnp.where(cnt0 >= kk, jnp.int32(0), jnp.int32(-2147483648))

    def step(t, P):
        bit = jnp.int32(1) << (jnp.int32(30) - t)
        cand = base + (P | bit)
        cnt = jnp.sum((k >= cand).astype(jnp.int32), axis=(1, 2),
                      keepdims=True)
        return jnp.where(cnt >= kk, P | bit, P)

    P = lax.fori_loop(0, 31, step, jnp.zeros_like(base))
    T = base + P

    m_gt = (k > T).astype(jnp.float32)
    m_eq = (k == T).astype(jnp.float32)
    c_gt = jnp.sum(m_gt, axis=(1, 2), keepdims=True)

    i128 = lax.broadcasted_iota(jnp.int32, (CHUNK, CHUNK), 0)
    j128 = lax.broadcasted_iota(jnp.int32, (CHUNK, CHUNK), 1)
    tri_incl = (i128 <= j128).astype(jnp.float32)
    inch = lax.broadcasted_iota(jnp.int32, (NCH, NCH), 0)
    jnch = lax.broadcasted_iota(jnp.int32, (NCH, NCH), 1)
    tri_excl = (inch < jnch).astype(jnp.float32)

    def row_cumsum(m):
        m2 = m.reshape(RBLK * NCH, CHUNK)
        cc = lax.dot_general(m2, tri_incl, (((1,), (0,)), ((), ())),
                             preferred_element_type=jnp.float32)
        cc = cc.reshape(RBLK, NCH, CHUNK)
        tot = cc[:, :, CHUNK - 1]
        pre = lax.dot_general(tot, tri_excl, (((1,), (0,)), ((), ())),
                              preferred_element_type=jnp.float32)
        return cc + pre[:, :, None]

    p_gt = row_cumsum(m_gt)
    p_eq = row_cumsum(m_eq)
    quota = jnp.float32(TOPK) - c_gt
    p_sel = p_gt + jnp.minimum(p_eq, quota)
    p_ref[...] = p_sel.astype(jnp.int32).reshape(RBLK * NCH, CHUNK)
    xo_ref[...] = x.reshape(RBLK * NCH, CHUNK)


def _tc_stage(x3):
    return pl.pallas_call(
        _tc_body,
        grid=(ROWS // RBLK,),
        in_specs=[pl.BlockSpec((RBLK, NCH, CHUNK), lambda i: (i, 0, 0))],
        out_specs=[
            pl.BlockSpec((RBLK * NCH, CHUNK), lambda i: (i, 0)),
            pl.BlockSpec((RBLK * NCH, CHUNK), lambda i: (i, 0)),
        ],
        out_shape=[
            jax.ShapeDtypeStruct((ROWS * NCH, CHUNK), jnp.int32),
            jax.ShapeDtypeStruct((ROWS * NCH, CHUNK), jnp.float32),
        ],
    )(x3)


def _sc_stage(p2, x2):
    mesh = plsc.VectorSubcoreMesh(core_axis_name="c", subcore_axis_name="s")

    @functools.partial(
        pl.kernel,
        mesh=mesh,
        compiler_params=pltpu.CompilerParams(needs_layout_passes=False),
        out_type=jax.ShapeDtypeStruct((ROWS, 2, 128), jnp.float32),
        scratch_types=[
            pltpu.VMEM((NCH, CHUNK), jnp.int32),
            pltpu.VMEM((NCH, CHUNK), jnp.float32),
            pltpu.VMEM((2, 128), jnp.float32),
            pltpu.SemaphoreType.DMA,
            pltpu.SemaphoreType.DMA,
        ],
    )
    def sc_kernel(p_hbm, x_hbm, out_hbm, p_v, x_v, val_v, psem, xsem):
        wid = lax.axis_index("s") * 2 + lax.axis_index("c")
        for rr in range(RPW):
            r = wid * RPW + rr
            pcopy = pltpu.async_copy(p_hbm.at[pl.ds(r * NCH, NCH)], p_v,
                                     psem)
            xcopy = pltpu.async_copy(x_hbm.at[pl.ds(r * NCH, NCH)], x_v,
                                     xsem)
            pcopy.wait()

            for h in range(2):
                def jbody(j, _, h=h):
                    t = (h * 128 + j * 16 + 1) + lax.iota(jnp.int32, 16)
                    pos = jnp.zeros(16, jnp.int32)
                    for sbit in range(14, -1, -1):
                        s = jnp.int32(1 << sbit)
                        probe = pos + (s - 1)
                        v = plsc.load_gather(
                            p_v, [probe >> 7, probe & 127])
                        pos = pos + jnp.where(v < t, s, jnp.int32(0))
                    vals = plsc.load_gather(x_v, [pos >> 7, pos & 127])
                    val_v[h, pl.ds(j * 16, 16)] = vals
                    return 0

                if h == 0:
                    xcopy.wait()
                lax.fori_loop(0, 8, jbody, 0)

            pltpu.sync_copy(val_v, out_hbm.at[r])

    return sc_kernel(p2, x2)


@jax.jit
def kernel(X):
    x3 = X.reshape(ROWS, NCH, CHUNK)
    p2, x2 = _tc_stage(x3)
    out = _sc_stage(p2, x2)
    return out.reshape(ROWS, TOPK)


# speedup vs baseline: 16.3247x; 1.1533x over previous
import functools

import jax
import jax.numpy as jnp
from jax import lax
from jax.experimental import pallas as pl
from jax.experimental.pallas import tpu as pltpu
from jax.experimental.pallas import tpu_sc as plsc

TOPK = 256
ROWS = 128
COLS = 32768
CHUNK = 128
NCH = COLS // CHUNK
RBLK = 16

NWORKERS = 32
RPW = ROWS // NWORKERS

def _to_key(v):
    b = lax.bitcast_convert_type(v, jnp.int32)
    return jnp.where(b < 0, b ^ jnp.int32(0x7FFFFFFF), b)


def _key_to_val(k):
    b = jnp.where(k < 0, k ^ jnp.int32(0x7FFFFFFF), k)
    return lax.bitcast_convert_type(b, jnp.float32)


def _tc_body(x_ref, p_ref, xo_ref):
    x = x_ref[...]
    k = _to_key(x)

    kk = jnp.int32(TOPK)
    nf = jnp.float32(1.0 / (NCH * CHUNK))

    def rsum(v):
        return jnp.sum(v, axis=(1, 2), keepdims=True)

    def count_ge(c):
        return rsum((k >= c).astype(jnp.int32))

    mu = rsum(x) * nf
    sigma = jnp.sqrt(jnp.maximum(rsum(x * x) * nf - mu * mu, 0.0))
    kmin_m1 = jnp.min(k, axis=(1, 2), keepdims=True) - 1
    kmax_p1 = jnp.max(k, axis=(1, 2), keepdims=True) + 1

    c1 = _to_key(mu + jnp.float32(2.30) * sigma)
    c2 = _to_key(mu + jnp.float32(2.55) * sigma)
    n1 = count_ge(c1)
    n2 = count_ge(c2)

    big_n = jnp.int32(NCH * CHUNK)
    lo = jnp.where(n2 >= kk, c2, jnp.where(n1 >= kk, c1, kmin_m1))
    clo = jnp.where(n2 >= kk, n2, jnp.where(n1 >= kk, n1, big_n))
    hi = jnp.where(n1 < kk, c1, jnp.where(n2 < kk, c2, kmax_p1))
    chi = jnp.where(n1 < kk, n1, jnp.where(n2 < kk, n2, jnp.int32(0)))

    def active_of(lo_, clo_, hi_):
        return (clo_ != kk) & (lo_ + 1 < hi_)

    def cond(carry):
        i, lo_, clo_, hi_, chi_ = carry
        return jnp.any(active_of(lo_, clo_, hi_))

    def body(carry):
        i, lo_, clo_, hi_, chi_ = carry
        d_f = hi_.astype(jnp.float32) - lo_.astype(jnp.float32)
        step_b = jnp.minimum(d_f * 0.5, jnp.float32(2.0e9))
        mid_b = lo_ + jnp.maximum(step_b.astype(jnp.int32), 1)
        lov = _key_to_val(lo_)
        hiv = _key_to_val(hi_)
        frac = ((clo_ - kk).astype(jnp.float32) + 0.5) / jnp.maximum(
            clo_ - chi_, 1).astype(jnp.float32)
        midv = lov + (hiv - lov) * frac
        ok_i = jnp.isfinite(midv) & (i % 2 == 0)
        mid = jnp.where(ok_i, _to_key(midv), mid_b)
        mid = jnp.minimum(jnp.maximum(mid, lo_ + 1), hi_ - 1)
        cm = count_ge(mid)
        act = active_of(lo_, clo_, hi_)
        take = cm >= kk
        lo_n = jnp.where(act & take, mid, lo_)
        clo_n = jnp.where(act & take, cm, clo_)
        hi_n = jnp.where(act & ~take, mid, hi_)
        chi_n = jnp.where(act & ~take, cm, chi_)
        return i + 1, lo_n, clo_n, hi_n, chi_n

    _, lo, clo, hi, chi = lax.while_loop(
        cond, body, (jnp.int32(0), lo, clo, hi, chi))
    T = lo

    m_gt = (k > T).astype(jnp.float32)
    m_eq = (k == T).astype(jnp.float32)
    c_gt = jnp.sum(m_gt, axis=(1, 2), keepdims=True)

    i128 = lax.broadcasted_iota(jnp.int32, (CHUNK, CHUNK), 0)
    j128 = lax.broadcasted_iota(jnp.int32, (CHUNK, CHUNK), 1)
    tri_incl = (i128 <= j128).astype(jnp.float32)
    inch = lax.broadcasted_iota(jnp.int32, (NCH, NCH), 0)
    jnch = lax.broadcasted_iota(jnp.int32, (NCH, NCH), 1)
    tri_excl = (inch < jnch).astype(jnp.float32)

    def row_cumsum(m):
        m2 = m.reshape(RBLK * NCH, CHUNK)
        cc = lax.dot_general(m2, tri_incl, (((1,), (0,)), ((), ())),
                             preferred_element_type=jnp.float32)
        cc = cc.reshape(RBLK, NCH, CHUNK)
        tot = cc[:, :, CHUNK - 1]
        pre = lax.dot_general(tot, tri_excl, (((1,), (0,)), ((), ())),
                              preferred_element_type=jnp.float32)
        return cc + pre[:, :, None]

    p_gt = row_cumsum(m_gt)
    p_eq = row_cumsum(m_eq)
    quota = jnp.float32(TOPK) - c_gt
    p_sel = p_gt + jnp.minimum(p_eq, quota)
    p_ref[...] = p_sel.astype(jnp.int32).reshape(RBLK * NCH, CHUNK)
    xo_ref[...] = x.reshape(RBLK * NCH, CHUNK)


def _tc_stage(x3):
    return pl.pallas_call(
        _tc_body,
        grid=(ROWS // RBLK,),
        in_specs=[pl.BlockSpec((RBLK, NCH, CHUNK), lambda i: (i, 0, 0))],
        out_specs=[
            pl.BlockSpec((RBLK * NCH, CHUNK), lambda i: (i, 0)),
            pl.BlockSpec((RBLK * NCH, CHUNK), lambda i: (i, 0)),
        ],
        out_shape=[
            jax.ShapeDtypeStruct((ROWS * NCH, CHUNK), jnp.int32),
            jax.ShapeDtypeStruct((ROWS * NCH, CHUNK), jnp.float32),
        ],
    )(x3)


def _sc_stage(p2, x2):
    mesh = plsc.VectorSubcoreMesh(core_axis_name="c", subcore_axis_name="s")

    @functools.partial(
        pl.kernel,
        mesh=mesh,
        compiler_params=pltpu.CompilerParams(needs_layout_passes=False),
        out_type=jax.ShapeDtypeStruct((ROWS, 2, 128), jnp.float32),
        scratch_types=[
            pltpu.VMEM((NCH, CHUNK), jnp.int32),
            pltpu.VMEM((NCH, CHUNK), jnp.float32),
            pltpu.VMEM((2, 128), jnp.float32),
            pltpu.SemaphoreType.DMA,
            pltpu.SemaphoreType.DMA,
        ],
    )
    def sc_kernel(p_hbm, x_hbm, out_hbm, p_v, x_v, val_v, psem, xsem):
        wid = lax.axis_index("s") * 2 + lax.axis_index("c")
        for rr in range(RPW):
            r = wid * RPW + rr
            pcopy = pltpu.async_copy(p_hbm.at[pl.ds(r * NCH, NCH)], p_v,
                                     psem)
            xcopy = pltpu.async_copy(x_hbm.at[pl.ds(r * NCH, NCH)], x_v,
                                     xsem)
            pcopy.wait()

            for h in range(2):
                def jbody(j, _, h=h):
                    t = (h * 128 + j * 16 + 1) + lax.iota(jnp.int32, 16)
                    pos = jnp.zeros(16, jnp.int32)
                    for sbit in range(14, -1, -1):
                        s = jnp.int32(1 << sbit)
                        probe = pos + (s - 1)
                        v = plsc.load_gather(
                            p_v, [probe >> 7, probe & 127])
                        pos = pos + jnp.where(v < t, s, jnp.int32(0))
                    vals = plsc.load_gather(x_v, [pos >> 7, pos & 127])
                    val_v[h, pl.ds(j * 16, 16)] = vals
                    return 0

                if h == 0:
                    xcopy.wait()
                lax.fori_loop(0, 8, jbody, 0)

            pltpu.sync_copy(val_v, out_hbm.at[r])

    return sc_kernel(p2, x2)


@jax.jit
def kernel(X):
    x3 = X.reshape(ROWS, NCH, CHUNK)
    p2, x2 = _tc_stage(x3)
    out = _sc_stage(p2, x2)
    return out.reshape(ROWS, TOPK)


# speedup vs baseline: 17.0910x; 1.0469x over previous
import functools

import jax
import jax.numpy as jnp
from jax import lax
from jax.experimental import pallas as pl
from jax.experimental.pallas import tpu as pltpu
from jax.experimental.pallas import tpu_sc as plsc

TOPK = 256
ROWS = 128
COLS = 32768
CHUNK = 128
NCH = COLS // CHUNK
RBLK = 16

NWORKERS = 32
RPW = ROWS // NWORKERS

def _to_key(v):
    b = lax.bitcast_convert_type(v, jnp.int32)
    return jnp.where(b < 0, b ^ jnp.int32(0x7FFFFFFF), b)


def _key_to_val(k):
    b = jnp.where(k < 0, k ^ jnp.int32(0x7FFFFFFF), k)
    return lax.bitcast_convert_type(b, jnp.float32)


def _tc_body(x_ref, p_ref, xo_ref):
    x = x_ref[...]
    k = _to_key(x)

    kk = jnp.int32(TOPK)

    def rsum(v):
        return jnp.sum(v, axis=(1, 2), keepdims=True)

    def count_ge(c):
        return rsum((k >= c).astype(jnp.int32))

    kmin_m1 = jnp.min(k, axis=(1, 2), keepdims=True) - 1
    kmax_p1 = jnp.max(k, axis=(1, 2), keepdims=True) + 1

    c1 = jnp.full_like(kmin_m1, _to_key(jnp.float32(2.30)))
    c2 = jnp.full_like(kmin_m1, _to_key(jnp.float32(2.55)))
    n1 = count_ge(c1)
    n2 = count_ge(c2)

    big_n = jnp.int32(NCH * CHUNK)
    lo = jnp.where(n2 >= kk, c2, jnp.where(n1 >= kk, c1, kmin_m1))
    clo = jnp.where(n2 >= kk, n2, jnp.where(n1 >= kk, n1, big_n))
    hi = jnp.where(n1 < kk, c1, jnp.where(n2 < kk, c2, kmax_p1))
    chi = jnp.where(n1 < kk, n1, jnp.where(n2 < kk, n2, jnp.int32(0)))

    def active_of(lo_, clo_, hi_):
        return (clo_ != kk) & (lo_ + 1 < hi_)

    def cond(carry):
        i, lo_, clo_, hi_, chi_ = carry
        return jnp.any(active_of(lo_, clo_, hi_))

    def body(carry):
        i, lo_, clo_, hi_, chi_ = carry
        d_f = hi_.astype(jnp.float32) - lo_.astype(jnp.float32)
        step_b = jnp.minimum(d_f * 0.5, jnp.float32(2.0e9))
        mid_b = lo_ + jnp.maximum(step_b.astype(jnp.int32), 1)
        lov = _key_to_val(lo_)
        hiv = _key_to_val(hi_)
        frac = ((clo_ - kk).astype(jnp.float32) + 0.5) / jnp.maximum(
            clo_ - chi_, 1).astype(jnp.float32)
        midv = lov + (hiv - lov) * frac
        ok_i = jnp.isfinite(midv) & (i % 2 == 0)
        mid = jnp.where(ok_i, _to_key(midv), mid_b)
        mid = jnp.minimum(jnp.maximum(mid, lo_ + 1), hi_ - 1)
        cm = count_ge(mid)
        act = active_of(lo_, clo_, hi_)
        take = cm >= kk
        lo_n = jnp.where(act & take, mid, lo_)
        clo_n = jnp.where(act & take, cm, clo_)
        hi_n = jnp.where(act & ~take, mid, hi_)
        chi_n = jnp.where(act & ~take, cm, chi_)
        return i + 1, lo_n, clo_n, hi_n, chi_n

    _, lo, clo, hi, chi = lax.while_loop(
        cond, body, (jnp.int32(0), lo, clo, hi, chi))
    T = lo

    i128 = lax.broadcasted_iota(jnp.int32, (CHUNK, CHUNK), 0)
    j128 = lax.broadcasted_iota(jnp.int32, (CHUNK, CHUNK), 1)
    tri_incl = (i128 <= j128).astype(jnp.float32)
    inch = lax.broadcasted_iota(jnp.int32, (NCH, NCH), 0)
    jnch = lax.broadcasted_iota(jnp.int32, (NCH, NCH), 1)
    tri_excl = (inch < jnch).astype(jnp.float32)

    def row_cumsum(m):
        m2 = m.reshape(RBLK * NCH, CHUNK)
        cc = lax.dot_general(m2, tri_incl, (((1,), (0,)), ((), ())),
                             preferred_element_type=jnp.float32)
        cc = cc.reshape(RBLK, NCH, CHUNK)
        tot = cc[:, :, CHUNK - 1]
        pre = lax.dot_general(tot, tri_excl, (((1,), (0,)), ((), ())),
                              preferred_element_type=jnp.float32)
        return cc + pre[:, :, None]

    def fast_path():
        m_ge = (k >= T).astype(jnp.float32)
        p_ref[...] = row_cumsum(m_ge).astype(jnp.int32).reshape(
            RBLK * NCH, CHUNK)

    def slow_path():
        m_gt = (k > T).astype(jnp.float32)
        m_eq = (k == T).astype(jnp.float32)
        c_gt = jnp.sum(m_gt, axis=(1, 2), keepdims=True)
        p_gt = row_cumsum(m_gt)
        p_eq = row_cumsum(m_eq)
        quota = jnp.float32(TOPK) - c_gt
        p_sel = p_gt + jnp.minimum(p_eq, quota)
        p_ref[...] = p_sel.astype(jnp.int32).reshape(RBLK * NCH, CHUNK)

    lax.cond(jnp.all(clo == kk), fast_path, slow_path)
    xo_ref[...] = x.reshape(RBLK * NCH, CHUNK)


def _tc_stage(x3):
    return pl.pallas_call(
        _tc_body,
        grid=(ROWS // RBLK,),
        in_specs=[pl.BlockSpec((RBLK, NCH, CHUNK), lambda i: (i, 0, 0))],
        out_specs=[
            pl.BlockSpec((RBLK * NCH, CHUNK), lambda i: (i, 0)),
            pl.BlockSpec((RBLK * NCH, CHUNK), lambda i: (i, 0)),
        ],
        out_shape=[
            jax.ShapeDtypeStruct((ROWS * NCH, CHUNK), jnp.int32),
            jax.ShapeDtypeStruct((ROWS * NCH, CHUNK), jnp.float32),
        ],
    )(x3)


def _sc_stage(p2, x2):
    mesh = plsc.VectorSubcoreMesh(core_axis_name="c", subcore_axis_name="s")

    @functools.partial(
        pl.kernel,
        mesh=mesh,
        compiler_params=pltpu.CompilerParams(needs_layout_passes=False),
        out_type=jax.ShapeDtypeStruct((ROWS, 2, 128), jnp.float32),
        scratch_types=[
            pltpu.VMEM((NCH, CHUNK), jnp.int32),
            pltpu.VMEM((NCH, CHUNK), jnp.float32),
            pltpu.VMEM((2, 128), jnp.float32),
            pltpu.SemaphoreType.DMA,
            pltpu.SemaphoreType.DMA,
        ],
    )
    def sc_kernel(p_hbm, x_hbm, out_hbm, p_v, x_v, val_v, psem, xsem):
        wid = lax.axis_index("s") * 2 + lax.axis_index("c")
        for rr in range(RPW):
            r = wid * RPW + rr
            pcopy = pltpu.async_copy(p_hbm.at[pl.ds(r * NCH, NCH)], p_v,
                                     psem)
            xcopy = pltpu.async_copy(x_hbm.at[pl.ds(r * NCH, NCH)], x_v,
                                     xsem)
            pcopy.wait()

            for h in range(2):
                def jbody(j, _, h=h):
                    t = (h * 128 + j * 16 + 1) + lax.iota(jnp.int32, 16)
                    pos = jnp.zeros(16, jnp.int32)
                    for sbit in range(14, -1, -1):
                        s = jnp.int32(1 << sbit)
                        probe = pos + (s - 1)
                        v = plsc.load_gather(
                            p_v, [probe >> 7, probe & 127])
                        pos = pos + jnp.where(v < t, s, jnp.int32(0))
                    vals = plsc.load_gather(x_v, [pos >> 7, pos & 127])
                    val_v[h, pl.ds(j * 16, 16)] = vals
                    return 0

                if h == 0:
                    xcopy.wait()
                lax.fori_loop(0, 8, jbody, 0)

            pltpu.sync_copy(val_v, out_hbm.at[r])

    return sc_kernel(p2, x2)


@jax.jit
def kernel(X):
    x3 = X.reshape(ROWS, NCH, CHUNK)
    p2, x2 = _tc_stage(x3)
    out = _sc_stage(p2, x2)
    return out.reshape(ROWS, TOPK)


# speedup vs baseline: 17.5813x; 1.0287x over previous
import functools

import jax
import jax.numpy as jnp
from jax import lax
from jax.experimental import pallas as pl
from jax.experimental.pallas import tpu as pltpu
from jax.experimental.pallas import tpu_sc as plsc

TOPK = 256
ROWS = 128
COLS = 32768
CHUNK = 128
NCH = COLS // CHUNK
RBLK = 16

NWORKERS = 32
RPW = ROWS // NWORKERS

def _to_key(v):
    b = lax.bitcast_convert_type(v, jnp.int32)
    return jnp.where(b < 0, b ^ jnp.int32(0x7FFFFFFF), b)


def _key_to_val(k):
    b = jnp.where(k < 0, k ^ jnp.int32(0x7FFFFFFF), k)
    return lax.bitcast_convert_type(b, jnp.float32)


def _tc_body(x_ref, p_ref, xo_ref):
    x = x_ref[...]
    k = _to_key(x)

    kk = jnp.int32(TOPK)

    def rsum(v):
        return jnp.sum(v, axis=(1, 2), keepdims=True)

    def count_ge(c):
        return rsum((k >= c).astype(jnp.int32))

    kmin_m1 = jnp.min(k, axis=(1, 2), keepdims=True) - 1
    kmax_p1 = jnp.max(k, axis=(1, 2), keepdims=True) + 1

    c1 = jnp.full_like(kmin_m1, _to_key(jnp.float32(2.30)))
    c2 = jnp.full_like(kmin_m1, _to_key(jnp.float32(2.55)))
    n1 = count_ge(c1)
    n2 = count_ge(c2)

    big_n = jnp.int32(NCH * CHUNK)
    lo = jnp.where(n2 >= kk, c2, jnp.where(n1 >= kk, c1, kmin_m1))
    clo = jnp.where(n2 >= kk, n2, jnp.where(n1 >= kk, n1, big_n))
    hi = jnp.where(n1 < kk, c1, jnp.where(n2 < kk, c2, kmax_p1))
    chi = jnp.where(n1 < kk, n1, jnp.where(n2 < kk, n2, jnp.int32(0)))

    def active_of(lo_, clo_, hi_):
        return (clo_ != kk) & (lo_ + 1 < hi_)

    def cond(carry):
        i, lo_, clo_, hi_, chi_ = carry
        return jnp.any(active_of(lo_, clo_, hi_))

    def body(carry):
        i, lo_, clo_, hi_, chi_ = carry
        d_f = hi_.astype(jnp.float32) - lo_.astype(jnp.float32)
        step_b = jnp.minimum(d_f * 0.5, jnp.float32(2.0e9))
        mid_b = lo_ + jnp.maximum(step_b.astype(jnp.int32), 1)
        lov = _key_to_val(lo_)
        hiv = _key_to_val(hi_)
        frac = ((clo_ - kk).astype(jnp.float32) + 0.5) / jnp.maximum(
            clo_ - chi_, 1).astype(jnp.float32)
        midv = lov + (hiv - lov) * frac
        ok_i = jnp.isfinite(midv) & (i % 2 == 0)
        mid = jnp.where(ok_i, _to_key(midv), mid_b)
        mid = jnp.minimum(jnp.maximum(mid, lo_ + 1), hi_ - 1)
        cm = count_ge(mid)
        act = active_of(lo_, clo_, hi_)
        take = cm >= kk
        lo_n = jnp.where(act & take, mid, lo_)
        clo_n = jnp.where(act & take, cm, clo_)
        hi_n = jnp.where(act & ~take, mid, hi_)
        chi_n = jnp.where(act & ~take, cm, chi_)
        return i + 1, lo_n, clo_n, hi_n, chi_n

    _, lo, clo, hi, chi = lax.while_loop(
        cond, body, (jnp.int32(0), lo, clo, hi, chi))
    T = lo

    i128 = lax.broadcasted_iota(jnp.int32, (CHUNK, CHUNK), 0)
    j128 = lax.broadcasted_iota(jnp.int32, (CHUNK, CHUNK), 1)
    tri_incl = (i128 <= j128).astype(jnp.float32)
    inch = lax.broadcasted_iota(jnp.int32, (NCH, NCH), 0)
    jnch = lax.broadcasted_iota(jnp.int32, (NCH, NCH), 1)
    tri_excl = (inch < jnch).astype(jnp.float32)

    def row_cumsum(m):
        m2 = m.reshape(RBLK * NCH, CHUNK)
        cc = lax.dot_general(m2, tri_incl, (((1,), (0,)), ((), ())),
                             preferred_element_type=jnp.float32)
        cc = cc.reshape(RBLK, NCH, CHUNK)
        tot = cc[:, :, CHUNK - 1]
        pre = lax.dot_general(tot, tri_excl, (((1,), (0,)), ((), ())),
                              preferred_element_type=jnp.float32)
        return cc + pre[:, :, None]

    def fast_path():
        m_ge = (k >= T).astype(jnp.float32)
        p_ref[...] = row_cumsum(m_ge).astype(jnp.int32).reshape(
            RBLK * NCH, CHUNK)

    def slow_path():
        m_gt = (k > T).astype(jnp.float32)
        m_eq = (k == T).astype(jnp.float32)
        c_gt = jnp.sum(m_gt, axis=(1, 2), keepdims=True)
        p_gt = row_cumsum(m_gt)
        p_eq = row_cumsum(m_eq)
        quota = jnp.float32(TOPK) - c_gt
        p_sel = p_gt + jnp.minimum(p_eq, quota)
        p_ref[...] = p_sel.astype(jnp.int32).reshape(RBLK * NCH, CHUNK)

    lax.cond(jnp.all(clo == kk), fast_path, slow_path)
    xo_ref[...] = x.reshape(RBLK * NCH, CHUNK)


def _tc_stage(x3, nrows):
    return pl.pallas_call(
        _tc_body,
        grid=(nrows // RBLK,),
        in_specs=[pl.BlockSpec((RBLK, NCH, CHUNK), lambda i: (i, 0, 0))],
        out_specs=[
            pl.BlockSpec((RBLK * NCH, CHUNK), lambda i: (i, 0)),
            pl.BlockSpec((RBLK * NCH, CHUNK), lambda i: (i, 0)),
        ],
        out_shape=[
            jax.ShapeDtypeStruct((nrows * NCH, CHUNK), jnp.int32),
            jax.ShapeDtypeStruct((nrows * NCH, CHUNK), jnp.float32),
        ],
    )(x3)


def _sc_stage(p2, x2, nrows):
    rpw = nrows // NWORKERS
    mesh = plsc.VectorSubcoreMesh(core_axis_name="c", subcore_axis_name="s")

    @functools.partial(
        pl.kernel,
        mesh=mesh,
        compiler_params=pltpu.CompilerParams(needs_layout_passes=False),
        out_type=jax.ShapeDtypeStruct((nrows, 2, 128), jnp.float32),
        scratch_types=[
            pltpu.VMEM((NCH, CHUNK), jnp.int32),
            pltpu.VMEM((NCH, CHUNK), jnp.float32),
            pltpu.VMEM((2, 128), jnp.float32),
            pltpu.SemaphoreType.DMA,
            pltpu.SemaphoreType.DMA,
        ],
    )
    def sc_kernel(p_hbm, x_hbm, out_hbm, p_v, x_v, val_v, psem, xsem):
        wid = lax.axis_index("s") * 2 + lax.axis_index("c")
        for rr in range(rpw):
            r = wid * rpw + rr
            pcopy = pltpu.async_copy(p_hbm.at[pl.ds(r * NCH, NCH)], p_v,
                                     psem)
            xcopy = pltpu.async_copy(x_hbm.at[pl.ds(r * NCH, NCH)], x_v,
                                     xsem)
            pcopy.wait()

            for h in range(2):
                def jbody(j, _, h=h):
                    t = (h * 128 + j * 16 + 1) + lax.iota(jnp.int32, 16)
                    pos = jnp.zeros(16, jnp.int32)
                    for sbit in range(14, -1, -1):
                        s = jnp.int32(1 << sbit)
                        probe = pos + (s - 1)
                        v = plsc.load_gather(
                            p_v, [probe >> 7, probe & 127])
                        pos = pos + jnp.where(v < t, s, jnp.int32(0))
                    vals = plsc.load_gather(x_v, [pos >> 7, pos & 127])
                    val_v[h, pl.ds(j * 16, 16)] = vals
                    return 0

                if h == 0:
                    xcopy.wait()
                lax.fori_loop(0, 8, jbody, 0)

            pltpu.sync_copy(val_v, out_hbm.at[r])

    return sc_kernel(p2, x2)


SPLITS = 2
RSPLIT = ROWS // SPLITS


@jax.jit
def kernel(X):
    outs = []
    for s in range(SPLITS):
        xs = X[s * RSPLIT:(s + 1) * RSPLIT].reshape(RSPLIT, NCH, CHUNK)
        p2, x2 = _tc_stage(xs, RSPLIT)
        outs.append(_sc_stage(p2, x2, RSPLIT))
    out = jnp.concatenate(outs, axis=0)
    return out.reshape(ROWS, TOPK)
